# Initial kernel scaffold; baseline (speedup 1.0000x reference)
import jax
import jax.numpy as jnp
from jax import lax
from jax.experimental import pallas as pl
from jax.experimental.pallas import tpu as pltpu
from jax.experimental.pallas import tpu_sc as plsc

B, S, D = 4096, 50, 64
BT = B * S
GROUP = 128
NG = BT // GROUP

_info = plsc.get_sparse_core_info()
NC, NS = _info.num_cores, _info.num_subcores
NW = NC * NS
GPW = NG // NW


def _sc_body(ids_hbm, tok_hbm, tr_hbm, pr_hbm, se_hbm, so_hbm, out_hbm,
             idx_t, idx_p, sc_e, sc_o, buf_t, buf_p, sem_t, sem_p):
    wid = lax.axis_index("s") * NC + lax.axis_index("c")
    g0 = wid * GPW
    pltpu.sync_copy(ids_hbm.at[pl.ds(g0, GPW)], idx_t)
    pltpu.sync_copy(tok_hbm.at[pl.ds(g0, GPW)], idx_p)
    pltpu.sync_copy(se_hbm.at[pl.ds(g0, GPW)], sc_e)
    pltpu.sync_copy(so_hbm.at[pl.ds(g0, GPW)], sc_o)

    def step(g, carry):
        cp_t = pltpu.async_copy(tr_hbm.at[idx_t.at[g]], buf_t, sem_t)
        cp_p = pltpu.async_copy(pr_hbm.at[idx_p.at[g]], buf_p, sem_p)
        cp_t.wait()
        cp_p.wait()
        pltpu.async_copy(buf_t, out_hbm.at[sc_e.at[g]], sem_t).wait()
        pltpu.async_copy(buf_p, out_hbm.at[sc_o.at[g]], sem_p).wait()
        return carry

    lax.fori_loop(0, GPW, step, 0)


_sc_call = pl.kernel(
    _sc_body,
    out_type=jax.ShapeDtypeStruct((2 * BT, D), jnp.float32),
    mesh=plsc.VectorSubcoreMesh(core_axis_name="c", subcore_axis_name="s"),
    scratch_types=[
        pltpu.VMEM((GPW, GROUP), jnp.int32),
        pltpu.VMEM((GPW, GROUP), jnp.int32),
        pltpu.VMEM((GPW, GROUP), jnp.int32),
        pltpu.VMEM((GPW, GROUP), jnp.int32),
        pltpu.VMEM((GROUP, D), jnp.float32),
        pltpu.VMEM((GROUP, D), jnp.float32),
        pltpu.SemaphoreType.DMA,
        pltpu.SemaphoreType.DMA,
    ],
)


@jax.jit
def _run(input_ids, tokens_pretrained, pretrained_table, trainable_table):
    ids = input_ids.reshape(NG, GROUP).astype(jnp.int32)
    tok = tokens_pretrained.reshape(NG, GROUP).astype(jnp.int32)
    flat = jnp.arange(BT, dtype=jnp.int32).reshape(NG, GROUP)
    se = flat * 2
    so = se + 1
    out2 = _sc_call(ids, tok, trainable_table, pretrained_table, se, so)
    return out2.reshape(B, S, 2 * D)


def kernel(input_ids, tokens_pretrained, pretrained_table, trainable_table):
    return _run(input_ids, tokens_pretrained, pretrained_table, trainable_table)


# speedup vs baseline: 1.1048x; 1.1048x over previous
import jax
import jax.numpy as jnp
from jax import lax
from jax.experimental import pallas as pl
from jax.experimental.pallas import tpu as pltpu
from jax.experimental.pallas import tpu_sc as plsc

B, S, D = 4096, 50, 64
BT = B * S
GROUP = 128
NG = BT // GROUP

_info = plsc.get_sparse_core_info()
NC, NS = _info.num_cores, _info.num_subcores
NW = NC * NS
GPW = NG // NW


def _sc_body(ids_hbm, tok_hbm, tr_hbm, pr_hbm, se_hbm, so_hbm, out_hbm,
             idx_t, idx_p, sc_e, sc_o, buf_t, buf_p, sem_t, sem_p):
    wid = lax.axis_index("s") * NC + lax.axis_index("c")
    pltpu.sync_copy(ids_hbm.at[wid], idx_t)
    pltpu.sync_copy(tok_hbm.at[wid], idx_p)
    pltpu.sync_copy(se_hbm.at[wid], sc_e)
    pltpu.sync_copy(so_hbm.at[wid], sc_o)

    def step(g, carry):
        cp_t = pltpu.async_copy(tr_hbm.at[idx_t.at[g]], buf_t, sem_t)
        cp_p = pltpu.async_copy(pr_hbm.at[idx_p.at[g]], buf_p, sem_p)
        cp_t.wait()
        cp_p.wait()
        pltpu.async_copy(buf_t, out_hbm.at[sc_e.at[g]], sem_t).wait()
        pltpu.async_copy(buf_p, out_hbm.at[sc_o.at[g]], sem_p).wait()
        return carry

    lax.fori_loop(0, GPW, step, 0)


_sc_call = pl.kernel(
    _sc_body,
    out_type=jax.ShapeDtypeStruct((2 * BT, D), jnp.float32),
    mesh=plsc.VectorSubcoreMesh(core_axis_name="c", subcore_axis_name="s"),
    scratch_types=[
        pltpu.VMEM((GPW, GROUP), jnp.int32),
        pltpu.VMEM((GPW, GROUP), jnp.int32),
        pltpu.VMEM((GPW, GROUP), jnp.int32),
        pltpu.VMEM((GPW, GROUP), jnp.int32),
        pltpu.VMEM((GROUP, D), jnp.float32),
        pltpu.VMEM((GROUP, D), jnp.float32),
        pltpu.SemaphoreType.DMA,
        pltpu.SemaphoreType.DMA,
    ],
    compiler_params=pltpu.CompilerParams(use_tc_tiling_on_sc=False),
)


@jax.jit
def _run(input_ids, tokens_pretrained, pretrained_table, trainable_table):
    ids = input_ids.reshape(NW, GPW, GROUP).astype(jnp.int32)
    tok = tokens_pretrained.reshape(NW, GPW, GROUP).astype(jnp.int32)
    flat = jnp.arange(BT, dtype=jnp.int32).reshape(NW, GPW, GROUP)
    se = flat * 2
    so = se + 1
    out2 = _sc_call(ids, tok, trainable_table, pretrained_table, se, so)
    return out2.reshape(B, S, 2 * D)


def kernel(input_ids, tokens_pretrained, pretrained_table, trainable_table):
    return _run(input_ids, tokens_pretrained, pretrained_table, trainable_table)
